# baseline (speedup 1.0000x reference)
import functools

import jax
import jax.numpy as jnp
from jax import lax
from jax.experimental import pallas as pl
from jax.experimental.pallas import tpu as pltpu
from jax.experimental.pallas import tpu_sc as plsc

_NC = 2
_NS = 16
_NW = _NC * _NS
_CH = 128


@functools.partial(jax.jit, static_argnames=("nch", "emb"))
def _sc_gather(idx, table, *, nch, emb):
    mesh = plsc.VectorSubcoreMesh(core_axis_name="c", subcore_axis_name="s")
    total = _NW * nch * _CH

    @functools.partial(
        pl.kernel,
        out_type=jax.ShapeDtypeStruct((total, emb), jnp.float32),
        mesh=mesh,
        scratch_types=[
            pltpu.VMEM((nch, _CH), jnp.int32),
            pltpu.VMEM((_CH, emb), jnp.float32),
            pltpu.SemaphoreType.DMA,
        ],
        compiler_params=pltpu.CompilerParams(use_tc_tiling_on_sc=False),
    )
    def body(idx_hbm, table_hbm, out_hbm, idx_v, rows_v, gsem):
        wid = lax.axis_index("s") * _NC + lax.axis_index("c")
        base = wid * (nch * _CH)
        pltpu.sync_copy(idx_hbm.at[wid], idx_v)

        @pl.loop(0, nch)
        def _(j):
            pltpu.async_copy(table_hbm.at[idx_v.at[j]], rows_v, gsem).wait()
            pltpu.sync_copy(rows_v, out_hbm.at[pl.ds(base + j * _CH, _CH)])

    return body(idx, table)


def kernel(sequences, embedding_weight):
    b, l = sequences.shape
    _, emb = embedding_weight.shape
    total = b * l
    nch = total // (_NW * _CH)
    idx = sequences.reshape(_NW, nch, _CH).astype(jnp.int32)
    out = _sc_gather(idx, embedding_weight, nch=nch, emb=emb)
    return out.reshape(b, l, emb)


# speedup vs baseline: 1.1168x; 1.1168x over previous
import functools

import jax
import jax.numpy as jnp
from jax import lax
from jax.experimental import pallas as pl
from jax.experimental.pallas import tpu as pltpu
from jax.experimental.pallas import tpu_sc as plsc

_NC = 2
_NS = 16
_NW = _NC * _NS
_CH = 128
_CPG = 2
_R = _CPG * _CH
_NBUF = 4
_AHEAD = 2


@functools.partial(jax.jit, static_argnames=("nch", "emb"))
def _sc_gather(idx, table, *, nch, emb):
    mesh = plsc.VectorSubcoreMesh(core_axis_name="c", subcore_axis_name="s")
    total = _NW * nch * _CH
    ng = nch // _CPG

    @functools.partial(
        pl.kernel,
        out_type=jax.ShapeDtypeStruct((total, emb), jnp.float32),
        mesh=mesh,
        scratch_types=[
            pltpu.VMEM((nch, _CH), jnp.int32),
            *[pltpu.VMEM((_R, emb), jnp.float32) for _ in range(_NBUF)],
            *[pltpu.SemaphoreType.DMA for _ in range(2 * _NBUF)],
        ],
        compiler_params=pltpu.CompilerParams(use_tc_tiling_on_sc=False),
    )
    def body(idx_hbm, table_hbm, out_hbm, idx_v, *bufs_and_sems):
        rows = bufs_and_sems[:_NBUF]
        gsems = bufs_and_sems[_NBUF:2 * _NBUF]
        ssems = bufs_and_sems[2 * _NBUF:]
        wid = lax.axis_index("s") * _NC + lax.axis_index("c")
        base = wid * (nch * _CH)

        def fire_gathers(g, b):
            for j in range(_CPG):
                pltpu.make_async_copy(
                    table_hbm.at[idx_v.at[g * _CPG + j]],
                    rows[b].at[pl.ds(j * _CH, _CH)],
                    gsems[b],
                ).start()

        def wait_gathers(g, b):
            for j in range(_CPG):
                pltpu.make_async_copy(
                    table_hbm.at[idx_v.at[g * _CPG + j]],
                    rows[b].at[pl.ds(j * _CH, _CH)],
                    gsems[b],
                ).wait()

        def store_desc(g, b):
            return pltpu.make_async_copy(
                rows[b], out_hbm.at[pl.ds(base + g * _R, _R)], ssems[b]
            )

        pltpu.sync_copy(idx_hbm.at[wid], idx_v)
        for p in range(_AHEAD):
            fire_gathers(p, p % _NBUF)

        @pl.loop(0, ng, step=_NBUF)
        def _(i):
            for k in range(_NBUF):
                g = i + k
                nxt = g + _AHEAD
                bcur = k
                bnxt = (k + _AHEAD) % _NBUF

                @pl.when(nxt < ng)
                def _():
                    @pl.when(nxt >= _NBUF)
                    def _():
                        store_desc(0, bnxt).wait()

                    fire_gathers(nxt, bnxt)

                wait_gathers(g, bcur)
                store_desc(g, bcur).start()

        for b in range(_NBUF):
            store_desc(0, b).wait()

    return body(idx, table)


def kernel(sequences, embedding_weight):
    b, l = sequences.shape
    _, emb = embedding_weight.shape
    total = b * l
    nch = total // (_NW * _CH)
    idx = sequences.reshape(_NW, nch, _CH).astype(jnp.int32)
    out = _sc_gather(idx, embedding_weight, nch=nch, emb=emb)
    return out.reshape(b, l, emb)
